# baseline (speedup 1.0000x reference)
import functools

import jax
import jax.numpy as jnp
from jax import lax
from jax.experimental import pallas as pl
from jax.experimental.pallas import tpu as pltpu
from jax.experimental.pallas import tpu_sc as plsc

_VOCAB = 100000
_D = 128
_BATCH = 4096
_HIST = 50
_TOT = _BATCH * _HIST

_NC = 2
_NS = 16
_NW = _NC * _NS
_BPW = _TOT // _NW

_C = 320
_NCH = _BPW // _C
_NHALF = _NCH // 2
_SPLITS = [(0, 128), (128, 128), (256, 64)]


def _gather_body(idx_hbm, table_hbm, out_hbm,
                 idx0_v, idx1_v, rows0_v, rows1_v,
                 isem0, isem1, gsem0, gsem1, osem0, osem1):
    wid = lax.axis_index("s") * _NC + lax.axis_index("c")
    base = wid * _BPW

    def idx_load(g, idx_v, sem):
        return pltpu.async_copy(
            idx_hbm.at[pl.ds(base + g * _C, _C)], idx_v, sem)

    def idx_drain(idx_v, sem):
        pltpu.make_async_copy(idx_hbm.at[pl.ds(0, _C)], idx_v, sem).wait()

    def fire_streams(idx_v, rows_v, sem):
        return [
            pltpu.async_copy(
                table_hbm.at[idx_v.at[pl.ds(s, n)]],
                rows_v.at[pl.ds(s, n)],
                sem,
            )
            for s, n in _SPLITS
        ]

    def drain_gather(rows_v, sem):
        pltpu.make_async_copy(table_hbm.at[pl.ds(0, _C)], rows_v, sem).wait()

    def pair(i, prefetch):
        g0 = 2 * i
        idx_drain(idx1_v, isem1)
        cps1 = fire_streams(idx1_v, rows1_v, gsem1)
        drain_gather(rows0_v, gsem0)
        if prefetch:
            icp0 = idx_load(g0 + 2, idx0_v, isem0)
        ocp0 = pltpu.async_copy(
            rows0_v, out_hbm.at[pl.ds(base + g0 * _C, _C)], osem0)
        for cp in cps1:
            cp.wait()
        ocp1 = pltpu.async_copy(
            rows1_v, out_hbm.at[pl.ds(base + (g0 + 1) * _C, _C)], osem1)
        if prefetch:
            idx_load(g0 + 3, idx1_v, isem1)
        ocp0.wait()
        if prefetch:
            icp0.wait()
            fire_streams(idx0_v, rows0_v, gsem0)
        ocp1.wait()

    idx_load(0, idx0_v, isem0).wait()
    fire_streams(idx0_v, rows0_v, gsem0)
    idx_load(1, idx1_v, isem1)

    @pl.loop(0, _NHALF - 1)
    def _pair(i):
        pair(i, prefetch=True)

    pair(_NHALF - 1, prefetch=False)


_sc_gather = functools.partial(
    pl.kernel,
    out_type=jax.ShapeDtypeStruct((_TOT, _D), jnp.float32),
    mesh=plsc.VectorSubcoreMesh(core_axis_name="c", subcore_axis_name="s"),
    scratch_types=[
        pltpu.VMEM((_C,), jnp.int32),
        pltpu.VMEM((_C,), jnp.int32),
        pltpu.VMEM((_C, _D), jnp.float32),
        pltpu.VMEM((_C, _D), jnp.float32),
        pltpu.SemaphoreType.DMA,
        pltpu.SemaphoreType.DMA,
        pltpu.SemaphoreType.DMA,
        pltpu.SemaphoreType.DMA,
        pltpu.SemaphoreType.DMA,
        pltpu.SemaphoreType.DMA,
    ],
)(_gather_body)


@jax.jit
def kernel(x, embeddings):
    idx = x.astype(jnp.int32).T.reshape(_TOT)
    out = _sc_gather(idx, embeddings)
    return out.reshape(_HIST, _BATCH, _D).transpose(1, 0, 2)


# speedup vs baseline: 1.0357x; 1.0357x over previous
import functools

import jax
import jax.numpy as jnp
from jax import lax
from jax.experimental import pallas as pl
from jax.experimental.pallas import tpu as pltpu
from jax.experimental.pallas import tpu_sc as plsc

_VOCAB = 100000
_D = 128
_BATCH = 4096
_HIST = 50
_TOT = _BATCH * _HIST

_NC = 2
_NS = 16
_NW = _NC * _NS
_BPW = _TOT // _NW

_C = 320
_NCH = _BPW // _C
_NHALF = _NCH // 2
_SPLITS = [(0, 128), (128, 128), (256, 64)]


def _gather_body(idx_hbm, table_hbm, out_hbm,
                 idx0_v, idx1_v, rows0_v, rows1_v,
                 gsem0, gsem1, osem0, osem1):
    wid = lax.axis_index("s") * _NC + lax.axis_index("c")
    base = wid * _BPW

    def fire_gather(g, idx_v, rows_v, sem):
        pltpu.sync_copy(idx_hbm.at[pl.ds(base + g * _C, _C)], idx_v)
        return [
            pltpu.async_copy(
                table_hbm.at[idx_v.at[pl.ds(s, n)]],
                rows_v.at[pl.ds(s, n)],
                sem,
            )
            for s, n in _SPLITS
        ]

    def drain_gather(rows_v, sem):
        pltpu.make_async_copy(table_hbm.at[pl.ds(0, _C)], rows_v, sem).wait()

    def pair(i, prefetch):
        g0 = 2 * i
        cps1 = fire_gather(g0 + 1, idx1_v, rows1_v, gsem1)
        drain_gather(rows0_v, gsem0)
        ocp0 = pltpu.async_copy(
            rows0_v, out_hbm.at[pl.ds(base + g0 * _C, _C)], osem0)
        for cp in cps1:
            cp.wait()
        ocp1 = pltpu.async_copy(
            rows1_v, out_hbm.at[pl.ds(base + (g0 + 1) * _C, _C)], osem1)
        ocp0.wait()
        if prefetch:
            fire_gather(g0 + 2, idx0_v, rows0_v, gsem0)
        ocp1.wait()

    fire_gather(0, idx0_v, rows0_v, gsem0)

    @pl.loop(0, _NHALF - 1)
    def _pair(i):
        pair(i, prefetch=True)

    pair(_NHALF - 1, prefetch=False)


_sc_gather = functools.partial(
    pl.kernel,
    out_type=jax.ShapeDtypeStruct((_TOT, _D), jnp.float32),
    mesh=plsc.VectorSubcoreMesh(core_axis_name="c", subcore_axis_name="s"),
    scratch_types=[
        pltpu.VMEM((_C,), jnp.int32),
        pltpu.VMEM((_C,), jnp.int32),
        pltpu.VMEM((_C, _D), jnp.float32),
        pltpu.VMEM((_C, _D), jnp.float32),
        pltpu.SemaphoreType.DMA,
        pltpu.SemaphoreType.DMA,
        pltpu.SemaphoreType.DMA,
        pltpu.SemaphoreType.DMA,
    ],
)(_gather_body)


@jax.jit
def kernel(x, embeddings):
    idx = x.astype(jnp.int32).T.reshape(_TOT)
    out = _sc_gather(idx, embeddings)
    return out.reshape(_HIST, _BATCH, _D).transpose(1, 0, 2)
